# Initial kernel scaffold; baseline (speedup 1.0000x reference)
import jax
import jax.numpy as jnp
from jax.experimental import pallas as pl


def kernel(x, up_index, up_orient, down_index, down_orient, batch, W_up_0, W_down_0, W_0, W_up_1, W_down_1, W_1, W_up_2, W_down_2, W_2, lin1_W, lin1_b, lin2_W, lin2_b):
    raise NotImplementedError("write your pallas kernel here")


# speedup vs baseline: 2.0846x; 2.0846x over previous
import functools
import jax
import jax.numpy as jnp
from jax import lax
from jax.experimental import pallas as pl
from jax.experimental.pallas import tpu as pltpu
from jax.experimental.pallas import tpu_sc as plsc

_N = 10000
_D = 128
_H = 128
_E = 320000
_B = 8

_NC = 2
_NS = 16
_NW = _NC * _NS
_K = 128
_EPW = 20480
_EP = _NW * _EPW
_ER = _E // 128
_PADR = _EP // 128 - 2 * _ER
_ROWS_PER_TILE = 624


def _idx_body(us_ref, uo_ref, ud_ref, ds_ref, do_ref, dd_ref,
              src2_ref, dst2_ref):
    su = us_ref[...] * 4 + (uo_ref[...] < 0).astype(jnp.int32)
    sd = ds_ref[...] * 4 + 2 + (do_ref[...] < 0).astype(jnp.int32)
    pad_s = jnp.zeros((_PADR, 128), jnp.int32)
    src2_ref[...] = jnp.concatenate([su, sd, pad_s], axis=0)
    pad_d = jnp.full((_PADR, 128), _N, jnp.int32)
    dst2_ref[...] = jnp.concatenate([ud_ref[...], dd_ref[...], pad_d], axis=0)


def _build_indices(up_src, up_orient, up_dst, down_src, down_orient, down_dst):
    out = pl.pallas_call(
        _idx_body,
        out_shape=(
            jax.ShapeDtypeStruct((_EP // 128, 128), jnp.int32),
            jax.ShapeDtypeStruct((_EP // 128, 128), jnp.int32),
        ),
    )(up_src.reshape(_ER, 128), up_orient.reshape(_ER, 128),
      up_dst.reshape(_ER, 128), down_src.reshape(_ER, 128),
      down_orient.reshape(_ER, 128), down_dst.reshape(_ER, 128))
    return out[0].reshape(_EP), out[1].reshape(_EP)


_RB = 2000


def _mm_body1(x_ref, wcat_ref, wt_ref, t_ref, hw_ref):
    xs = x_ref[...]
    t_ref[...] = jnp.dot(xs, wcat_ref[...], preferred_element_type=jnp.float32)
    hw_ref[...] = jnp.dot(xs, wt_ref[...], preferred_element_type=jnp.float32)


def _mm_body3(a0_ref, a1_ref, hwp_ref, wcat_ref, wt_ref, t_ref, hw_ref):
    xs = a0_ref[...] + a1_ref[...] + hwp_ref[...]
    t_ref[...] = jnp.dot(xs, wcat_ref[...], preferred_element_type=jnp.float32)
    hw_ref[...] = jnp.dot(xs, wt_ref[...], preferred_element_type=jnp.float32)


def _layer_matmul(terms, wcat_t, w_t):
    nterms = len(terms)
    body = _mm_body1 if nterms == 1 else _mm_body3
    row_spec = pl.BlockSpec((_RB, 128), lambda i: (i, 0))
    in_specs = [row_spec] * nterms + [
        pl.BlockSpec((128, 512), lambda i: (0, 0)),
        pl.BlockSpec((128, 128), lambda i: (0, 0)),
    ]
    t_raw, hw = pl.pallas_call(
        body,
        grid=(_N // _RB,),
        in_specs=in_specs,
        out_specs=(
            pl.BlockSpec((_RB, 512), lambda i: (i, 0)),
            pl.BlockSpec((_RB, 128), lambda i: (i, 0)),
        ),
        out_shape=(
            jax.ShapeDtypeStruct((_N, 512), jnp.float32),
            jax.ShapeDtypeStruct((_N, 128), jnp.float32),
        ),
    )(*terms, wcat_t, w_t)
    return t_raw.reshape(4 * _N, 128), hw


_SC_MESH = plsc.VectorSubcoreMesh(core_axis_name="c", subcore_axis_name="s")


@functools.partial(
    pl.kernel,
    out_type=jax.ShapeDtypeStruct((_NC, _N, 128), jnp.float32),
    mesh=_SC_MESH,
    scratch_types=[
        pltpu.VMEM((_K,), jnp.int32),
        pltpu.VMEM((_K,), jnp.int32),
        pltpu.VMEM((_K, 128), jnp.float32),
        pltpu.VMEM_SHARED((_N + 16, 128), jnp.float32),
        pltpu.SemaphoreType.DMA,
    ],
)
def _sc_scatter(table_hbm, src2_hbm, dst2_hbm, zeros_hbm, out_hbm,
                sidx, didx, rows, acc, gsem):
    c = lax.axis_index("c")
    s = lax.axis_index("s")
    w = c * _NS + s
    r0 = s * _ROWS_PER_TILE
    tail = _NS * _ROWS_PER_TILE
    pltpu.sync_copy(zeros_hbm.at[pl.ds(r0, _ROWS_PER_TILE)],
                    acc.at[pl.ds(r0, _ROWS_PER_TILE)])

    @pl.when(s == _NS - 1)
    def _():
        pltpu.sync_copy(zeros_hbm.at[pl.ds(tail, _N - tail)],
                        acc.at[pl.ds(tail, _N - tail)])

    plsc.subcore_barrier()

    def body(i, carry):
        base = w * _EPW + i * _K
        pltpu.sync_copy(src2_hbm.at[pl.ds(base, _K)], sidx)
        pltpu.sync_copy(dst2_hbm.at[pl.ds(base, _K)], didx)
        pltpu.async_copy(table_hbm.at[sidx], rows, gsem).wait()
        pltpu.sync_copy(rows, acc.at[didx], add=True)
        return carry

    lax.fori_loop(0, _EPW // _K, body, 0)
    plsc.subcore_barrier()
    pltpu.sync_copy(acc.at[pl.ds(r0, _ROWS_PER_TILE)],
                    out_hbm.at[c, pl.ds(r0, _ROWS_PER_TILE)])

    @pl.when(s == _NS - 1)
    def _():
        pltpu.sync_copy(acc.at[pl.ds(tail, _N - tail)],
                        out_hbm.at[c, pl.ds(tail, _N - tail)])


def _pool_body(a0_ref, a1_ref, hw_ref, bt_ref, w1_ref, b1_ref, w2_ref, b2_ref,
               out_ref, pooled_ref):
    i = pl.program_id(0)
    h = jnp.abs(a0_ref[...] + a1_ref[...] + hw_ref[...])
    onehot = (bt_ref[...] == lax.broadcasted_iota(jnp.int32, (1, _B), 1)
              ).astype(jnp.float32)
    part = lax.dot_general(onehot, h, (((0,), (0,)), ((), ())),
                           preferred_element_type=jnp.float32)

    @pl.when(i == 0)
    def _():
        pooled_ref[...] = part

    @pl.when(i > 0)
    def _():
        pooled_ref[...] += part

    @pl.when(i == _N // _RB - 1)
    def _():
        p = pooled_ref[...]
        h1 = jnp.maximum(
            jnp.dot(p, w1_ref[...], preferred_element_type=jnp.float32)
            + b1_ref[...], 0.0)
        out_ref[...] = jnp.dot(h1, w2_ref[...],
                               preferred_element_type=jnp.float32) + b2_ref[...]


def _pool_mlp(a0, a1, hw, batch2d, w1t, b1, w2t, b2):
    row_spec = pl.BlockSpec((_RB, 128), lambda i: (i, 0))
    const = lambda shape: pl.BlockSpec(shape, lambda i: (0, 0))
    return pl.pallas_call(
        _pool_body,
        grid=(_N // _RB,),
        in_specs=[row_spec, row_spec, row_spec,
                  pl.BlockSpec((_RB, 1), lambda i: (i, 0)),
                  const((128, 128)), const((1, 128)),
                  const((128, 128)), const((1, 128))],
        out_specs=const((_B, 128)),
        out_shape=jax.ShapeDtypeStruct((_B, 128), jnp.float32),
        scratch_shapes=[pltpu.VMEM((_B, 128), jnp.float32)],
    )(a0, a1, hw, batch2d, w1t, b1, w2t, b2)


def kernel(x, up_index, up_orient, down_index, down_orient, batch,
           W_up_0, W_down_0, W_0, W_up_1, W_down_1, W_1, W_up_2, W_down_2, W_2,
           lin1_W, lin1_b, lin2_W, lin2_b):
    f32 = jnp.float32
    src2, dst2 = _build_indices(up_index[0], up_orient, up_index[1],
                                down_index[0], down_orient, down_index[1])
    zeros = jnp.zeros((_N, 128), f32)

    def wcat(Wu, Wd):
        return jnp.concatenate([Wu.T, -Wu.T, Wd.T, -Wd.T], axis=1)

    layers = ((W_up_0, W_down_0, W_0), (W_up_1, W_down_1, W_1),
              (W_up_2, W_down_2, W_2))

    terms = (x,)
    hw = None
    for Wu, Wd, W in layers:
        t_tab, hw = _layer_matmul(terms, wcat(Wu, Wd), W.T)
        acc = _sc_scatter(t_tab, src2, dst2, zeros)
        terms = (acc[0], acc[1], hw)

    w2p = jnp.zeros((128, 128), f32).at[:, :2].set(lin2_W.T)
    b2p = jnp.zeros((1, 128), f32).at[0, :2].set(lin2_b)
    out = _pool_mlp(terms[0], terms[1], terms[2], batch.reshape(_N, 1),
                    lin1_W.T, lin1_b.reshape(1, 128), w2p, b2p)
    return out[:, :2]


# speedup vs baseline: 2.4121x; 1.1571x over previous
import functools
import jax
import jax.numpy as jnp
from jax import lax
from jax.experimental import pallas as pl
from jax.experimental.pallas import tpu as pltpu
from jax.experimental.pallas import tpu_sc as plsc

_N = 10000
_D = 128
_H = 128
_E = 320000
_B = 8

_NC = 2
_NS = 16
_NW = _NC * _NS
_K = 64
_EPW = 20480
_EP = _NW * _EPW
_ER = _E // 128
_PADR = _EP // 128 - 2 * _ER
_ROWS_PER_TILE = 624


def _idx_body(us_ref, uo_ref, ud_ref, ds_ref, do_ref, dd_ref,
              src2_ref, dst2_ref):
    su = us_ref[...] * 4 + (uo_ref[...] < 0).astype(jnp.int32)
    sd = ds_ref[...] * 4 + 2 + (do_ref[...] < 0).astype(jnp.int32)
    pad_s = jnp.zeros((_PADR, 128), jnp.int32)
    src2_ref[...] = jnp.concatenate([su, sd, pad_s], axis=0)
    pad_d = jnp.full((_PADR, 128), _N, jnp.int32)
    dst2_ref[...] = jnp.concatenate([ud_ref[...], dd_ref[...], pad_d], axis=0)


def _build_indices(up_src, up_orient, up_dst, down_src, down_orient, down_dst):
    out = pl.pallas_call(
        _idx_body,
        out_shape=(
            jax.ShapeDtypeStruct((_EP // 128, 128), jnp.int32),
            jax.ShapeDtypeStruct((_EP // 128, 128), jnp.int32),
        ),
    )(up_src.reshape(_ER, 128), up_orient.reshape(_ER, 128),
      up_dst.reshape(_ER, 128), down_src.reshape(_ER, 128),
      down_orient.reshape(_ER, 128), down_dst.reshape(_ER, 128))
    return out[0].reshape(_EP), out[1].reshape(_EP)


_RB = 2000


def _mm_body1(x_ref, wcat_ref, wt_ref, t_ref, hw_ref):
    xs = x_ref[...]
    t_ref[...] = jnp.dot(xs, wcat_ref[...], preferred_element_type=jnp.float32)
    hw_ref[...] = jnp.dot(xs, wt_ref[...], preferred_element_type=jnp.float32)


def _mm_body3(a0_ref, a1_ref, hwp_ref, wcat_ref, wt_ref, t_ref, hw_ref):
    xs = a0_ref[...] + a1_ref[...] + hwp_ref[...]
    t_ref[...] = jnp.dot(xs, wcat_ref[...], preferred_element_type=jnp.float32)
    hw_ref[...] = jnp.dot(xs, wt_ref[...], preferred_element_type=jnp.float32)


def _layer_matmul(terms, wcat_t, w_t):
    nterms = len(terms)
    body = _mm_body1 if nterms == 1 else _mm_body3
    row_spec = pl.BlockSpec((_RB, 128), lambda i: (i, 0))
    in_specs = [row_spec] * nterms + [
        pl.BlockSpec((128, 512), lambda i: (0, 0)),
        pl.BlockSpec((128, 128), lambda i: (0, 0)),
    ]
    t_raw, hw = pl.pallas_call(
        body,
        grid=(_N // _RB,),
        in_specs=in_specs,
        out_specs=(
            pl.BlockSpec((_RB, 512), lambda i: (i, 0)),
            pl.BlockSpec((_RB, 128), lambda i: (i, 0)),
        ),
        out_shape=(
            jax.ShapeDtypeStruct((_N, 512), jnp.float32),
            jax.ShapeDtypeStruct((_N, 128), jnp.float32),
        ),
    )(*terms, wcat_t, w_t)
    return t_raw.reshape(4 * _N, 128), hw


_SC_MESH = plsc.VectorSubcoreMesh(core_axis_name="c", subcore_axis_name="s")
_NB = 4
_NCHUNK = _EPW // _K
_NROUND = _NCHUNK // _NB


@functools.partial(
    pl.kernel,
    out_type=jax.ShapeDtypeStruct((_NC, _N, 128), jnp.float32),
    mesh=_SC_MESH,
    scratch_types=(
        [pltpu.VMEM((_K,), jnp.int32)] * _NB +
        [pltpu.VMEM((_K,), jnp.int32)] * _NB +
        [pltpu.VMEM((_K, 128), jnp.float32)] * _NB +
        [pltpu.VMEM_SHARED((_N + 16, 128), jnp.float32)] +
        [pltpu.SemaphoreType.DMA] * (2 * _NB)
    ),
)
def _sc_scatter(table_hbm, src2_hbm, dst2_hbm, zeros_hbm, out_hbm, *scr):
    sidx = scr[0:_NB]
    didx = scr[_NB:2 * _NB]
    rows = scr[2 * _NB:3 * _NB]
    acc = scr[3 * _NB]
    gsem = scr[3 * _NB + 1:3 * _NB + 1 + _NB]
    ssem = scr[3 * _NB + 1 + _NB:]
    c = lax.axis_index("c")
    s = lax.axis_index("s")
    w = c * _NS + s
    base_w = w * _EPW
    r0 = s * _ROWS_PER_TILE
    tail = _NS * _ROWS_PER_TILE
    pltpu.sync_copy(zeros_hbm.at[pl.ds(r0, _ROWS_PER_TILE)],
                    acc.at[pl.ds(r0, _ROWS_PER_TILE)])

    @pl.when(s == _NS - 1)
    def _():
        pltpu.sync_copy(zeros_hbm.at[pl.ds(tail, _N - tail)],
                        acc.at[pl.ds(tail, _N - tail)])

    plsc.subcore_barrier()

    def fetch(chunk, b):
        base = base_w + chunk * _K
        pltpu.sync_copy(src2_hbm.at[pl.ds(base, _K)], sidx[b])
        pltpu.sync_copy(dst2_hbm.at[pl.ds(base, _K)], didx[b])
        pltpu.async_copy(table_hbm.at[sidx[b]], rows[b], gsem[b])

    for b in range(_NB):
        fetch(b, b)

    def round_body(g, carry):
        for b in range(_NB):
            pltpu.make_async_copy(table_hbm.at[sidx[b]], rows[b],
                                  gsem[b]).wait()
            pltpu.async_copy(rows[b], acc.at[didx[b]], ssem[b], add=True)
        for b in range(_NB):
            @pl.when(g < _NROUND - 1)
            def _(b=b):
                pltpu.make_async_copy(rows[b], acc.at[didx[b]],
                                      ssem[b]).wait()
                fetch((g + 1) * _NB + b, b)
        return carry

    lax.fori_loop(0, _NROUND, round_body, 0)
    for b in range(_NB):
        pltpu.make_async_copy(rows[b], acc.at[didx[b]], ssem[b]).wait()
    plsc.subcore_barrier()
    pltpu.sync_copy(acc.at[pl.ds(r0, _ROWS_PER_TILE)],
                    out_hbm.at[c, pl.ds(r0, _ROWS_PER_TILE)])

    @pl.when(s == _NS - 1)
    def _():
        pltpu.sync_copy(acc.at[pl.ds(tail, _N - tail)],
                        out_hbm.at[c, pl.ds(tail, _N - tail)])


def _pool_body(a0_ref, a1_ref, hw_ref, bt_ref, w1_ref, b1_ref, w2_ref, b2_ref,
               out_ref, pooled_ref):
    i = pl.program_id(0)
    h = jnp.abs(a0_ref[...] + a1_ref[...] + hw_ref[...])
    onehot = (bt_ref[...] == lax.broadcasted_iota(jnp.int32, (1, _B), 1)
              ).astype(jnp.float32)
    part = lax.dot_general(onehot, h, (((0,), (0,)), ((), ())),
                           preferred_element_type=jnp.float32)

    @pl.when(i == 0)
    def _():
        pooled_ref[...] = part

    @pl.when(i > 0)
    def _():
        pooled_ref[...] += part

    @pl.when(i == _N // _RB - 1)
    def _():
        p = pooled_ref[...]
        h1 = jnp.maximum(
            jnp.dot(p, w1_ref[...], preferred_element_type=jnp.float32)
            + b1_ref[...], 0.0)
        out_ref[...] = jnp.dot(h1, w2_ref[...],
                               preferred_element_type=jnp.float32) + b2_ref[...]


def _pool_mlp(a0, a1, hw, batch2d, w1t, b1, w2t, b2):
    row_spec = pl.BlockSpec((_RB, 128), lambda i: (i, 0))
    const = lambda shape: pl.BlockSpec(shape, lambda i: (0, 0))
    return pl.pallas_call(
        _pool_body,
        grid=(_N // _RB,),
        in_specs=[row_spec, row_spec, row_spec,
                  pl.BlockSpec((_RB, 1), lambda i: (i, 0)),
                  const((128, 128)), const((1, 128)),
                  const((128, 128)), const((1, 128))],
        out_specs=const((_B, 128)),
        out_shape=jax.ShapeDtypeStruct((_B, 128), jnp.float32),
        scratch_shapes=[pltpu.VMEM((_B, 128), jnp.float32)],
    )(a0, a1, hw, batch2d, w1t, b1, w2t, b2)


def kernel(x, up_index, up_orient, down_index, down_orient, batch,
           W_up_0, W_down_0, W_0, W_up_1, W_down_1, W_1, W_up_2, W_down_2, W_2,
           lin1_W, lin1_b, lin2_W, lin2_b):
    f32 = jnp.float32
    src2, dst2 = _build_indices(up_index[0], up_orient, up_index[1],
                                down_index[0], down_orient, down_index[1])
    zeros = jnp.zeros((_N, 128), f32)

    def wcat(Wu, Wd):
        return jnp.concatenate([Wu.T, -Wu.T, Wd.T, -Wd.T], axis=1)

    layers = ((W_up_0, W_down_0, W_0), (W_up_1, W_down_1, W_1),
              (W_up_2, W_down_2, W_2))

    terms = (x,)
    hw = None
    for Wu, Wd, W in layers:
        t_tab, hw = _layer_matmul(terms, wcat(Wu, Wd), W.T)
        acc = _sc_scatter(t_tab, src2, dst2, zeros)
        terms = (acc[0], acc[1], hw)

    w2p = jnp.zeros((128, 128), f32).at[:, :2].set(lin2_W.T)
    b2p = jnp.zeros((1, 128), f32).at[0, :2].set(lin2_b)
    out = _pool_mlp(terms[0], terms[1], terms[2], batch.reshape(_N, 1),
                    lin1_W.T, lin1_b.reshape(1, 128), w2p, b2p)
    return out[:, :2]


# speedup vs baseline: 2.7644x; 1.1461x over previous
import functools
import jax
import jax.numpy as jnp
from jax import lax
from jax.experimental import pallas as pl
from jax.experimental.pallas import tpu as pltpu
from jax.experimental.pallas import tpu_sc as plsc

_N = 10000
_D = 128
_H = 128
_E = 320000
_B = 8

_NC = 2
_NS = 16
_NW = _NC * _NS
_K = 128
_EPW = 20480
_EP = _NW * _EPW
_ER = _E // 128
_PADR = _EP // 128 - 2 * _ER
_ROWS_PER_TILE = 624


def _idx_body(us_ref, uo_ref, ud_ref, ds_ref, do_ref, dd_ref, idx2_ref):
    su = us_ref[...] * 4 + (uo_ref[...] < 0).astype(jnp.int32)
    sd = ds_ref[...] * 4 + 2 + (do_ref[...] < 0).astype(jnp.int32)
    src_all = jnp.concatenate([su, sd, jnp.zeros((_PADR, 128), jnp.int32)],
                              axis=0)
    dst_all = jnp.concatenate([ud_ref[...], dd_ref[...],
                               jnp.full((_PADR, 128), _N, jnp.int32)], axis=0)
    idx2_ref[...] = jnp.stack([src_all, dst_all], axis=1)


def _build_indices(up_src, up_orient, up_dst, down_src, down_orient, down_dst):
    out = pl.pallas_call(
        _idx_body,
        out_shape=jax.ShapeDtypeStruct((_EP // _K, 2, _K), jnp.int32),
    )(up_src.reshape(_ER, 128), up_orient.reshape(_ER, 128),
      up_dst.reshape(_ER, 128), down_src.reshape(_ER, 128),
      down_orient.reshape(_ER, 128), down_dst.reshape(_ER, 128))
    return out.reshape(2 * _EP // _K, _K)


_RB = 2000


def _mm_body1(x_ref, wcat_ref, wt_ref, t_ref, hw_ref):
    xs = x_ref[...]
    t_ref[...] = jnp.dot(xs, wcat_ref[...], preferred_element_type=jnp.float32)
    hw_ref[...] = jnp.dot(xs, wt_ref[...], preferred_element_type=jnp.float32)


def _mm_body3(a0_ref, a1_ref, hwp_ref, wcat_ref, wt_ref, t_ref, hw_ref):
    xs = a0_ref[...] + a1_ref[...] + hwp_ref[...]
    t_ref[...] = jnp.dot(xs, wcat_ref[...], preferred_element_type=jnp.float32)
    hw_ref[...] = jnp.dot(xs, wt_ref[...], preferred_element_type=jnp.float32)


def _layer_matmul(terms, wcat_t, w_t):
    nterms = len(terms)
    body = _mm_body1 if nterms == 1 else _mm_body3
    row_spec = pl.BlockSpec((_RB, 128), lambda i: (i, 0))
    in_specs = [row_spec] * nterms + [
        pl.BlockSpec((128, 512), lambda i: (0, 0)),
        pl.BlockSpec((128, 128), lambda i: (0, 0)),
    ]
    t_raw, hw = pl.pallas_call(
        body,
        grid=(_N // _RB,),
        in_specs=in_specs,
        out_specs=(
            pl.BlockSpec((_RB, 512), lambda i: (i, 0)),
            pl.BlockSpec((_RB, 128), lambda i: (i, 0)),
        ),
        out_shape=(
            jax.ShapeDtypeStruct((_N, 512), jnp.float32),
            jax.ShapeDtypeStruct((_N, 128), jnp.float32),
        ),
    )(*terms, wcat_t, w_t)
    return t_raw.reshape(4 * _N, 128), hw


_SC_MESH = plsc.VectorSubcoreMesh(core_axis_name="c", subcore_axis_name="s")
_NB = 2
_NCHUNK = _EPW // _K
_NROUND = _NCHUNK // _NB


@functools.partial(
    pl.kernel,
    out_type=jax.ShapeDtypeStruct((_NC, _N, 128), jnp.float32),
    mesh=_SC_MESH,
    scratch_types=(
        [pltpu.VMEM((2 * _NB, _K), jnp.int32)] * 2 +
        [pltpu.VMEM((_K, 128), jnp.float32)] * _NB +
        [pltpu.VMEM_SHARED((_N + 16, 128), jnp.float32)] +
        [pltpu.SemaphoreType.DMA] * (2 + 2 * _NB)
    ),
)
def _sc_scatter(table_hbm, idx2_hbm, zeros_hbm, out_hbm, *scr):
    ibufs = scr[0:2]
    rows = scr[2:2 + _NB]
    acc = scr[2 + _NB]
    isems = scr[3 + _NB:5 + _NB]
    gsems = scr[5 + _NB:5 + 2 * _NB]
    ssems = scr[5 + 2 * _NB:5 + 3 * _NB]
    c = lax.axis_index("c")
    s = lax.axis_index("s")
    w = c * _NS + s
    cbase = w * _NCHUNK
    r0 = s * _ROWS_PER_TILE
    tail = _NS * _ROWS_PER_TILE
    pltpu.sync_copy(zeros_hbm.at[pl.ds(r0, _ROWS_PER_TILE)],
                    acc.at[pl.ds(r0, _ROWS_PER_TILE)])

    @pl.when(s == _NS - 1)
    def _():
        pltpu.sync_copy(zeros_hbm.at[pl.ds(tail, _N - tail)],
                        acc.at[pl.ds(tail, _N - tail)])

    plsc.subcore_barrier()

    def ifetch(q, d):
        pltpu.async_copy(idx2_hbm.at[pl.ds(2 * (cbase + _NB * q), 2 * _NB)],
                         ibufs[d], isems[d])

    ifetch(0, 0)
    ifetch(1, 1)

    def sub_round(q, d):
        pltpu.make_async_copy(
            idx2_hbm.at[pl.ds(2 * (cbase + _NB * q), 2 * _NB)],
            ibufs[d], isems[d]).wait()
        gd = [pltpu.async_copy(table_hbm.at[ibufs[d].at[2 * b]], rows[b],
                               gsems[b]) for b in range(_NB)]
        sd = []
        for b in range(_NB):
            gd[b].wait()
            sd.append(pltpu.async_copy(rows[b],
                                       acc.at[ibufs[d].at[2 * b + 1]],
                                       ssems[b], add=True))
        for b in range(_NB):
            sd[b].wait()

        @pl.when(q < _NROUND - 2)
        def _():
            ifetch(q + 2, d)

    def pair_body(p, carry):
        sub_round(2 * p, 0)
        sub_round(2 * p + 1, 1)
        return carry

    lax.fori_loop(0, _NROUND // 2, pair_body, 0)
    plsc.subcore_barrier()
    pltpu.sync_copy(acc.at[pl.ds(r0, _ROWS_PER_TILE)],
                    out_hbm.at[c, pl.ds(r0, _ROWS_PER_TILE)])

    @pl.when(s == _NS - 1)
    def _():
        pltpu.sync_copy(acc.at[pl.ds(tail, _N - tail)],
                        out_hbm.at[c, pl.ds(tail, _N - tail)])


def _pool_body(a0_ref, a1_ref, hw_ref, bt_ref, w1_ref, b1_ref, w2_ref, b2_ref,
               out_ref, pooled_ref):
    i = pl.program_id(0)
    h = jnp.abs(a0_ref[...] + a1_ref[...] + hw_ref[...])
    onehot = (bt_ref[...] == lax.broadcasted_iota(jnp.int32, (1, _B), 1)
              ).astype(jnp.float32)
    part = lax.dot_general(onehot, h, (((0,), (0,)), ((), ())),
                           preferred_element_type=jnp.float32)

    @pl.when(i == 0)
    def _():
        pooled_ref[...] = part

    @pl.when(i > 0)
    def _():
        pooled_ref[...] += part

    @pl.when(i == _N // _RB - 1)
    def _():
        p = pooled_ref[...]
        h1 = jnp.maximum(
            jnp.dot(p, w1_ref[...], preferred_element_type=jnp.float32)
            + b1_ref[...], 0.0)
        out_ref[...] = jnp.dot(h1, w2_ref[...],
                               preferred_element_type=jnp.float32) + b2_ref[...]


def _pool_mlp(a0, a1, hw, batch2d, w1t, b1, w2t, b2):
    row_spec = pl.BlockSpec((_RB, 128), lambda i: (i, 0))
    const = lambda shape: pl.BlockSpec(shape, lambda i: (0, 0))
    return pl.pallas_call(
        _pool_body,
        grid=(_N // _RB,),
        in_specs=[row_spec, row_spec, row_spec,
                  pl.BlockSpec((_RB, 1), lambda i: (i, 0)),
                  const((128, 128)), const((1, 128)),
                  const((128, 128)), const((1, 128))],
        out_specs=const((_B, 128)),
        out_shape=jax.ShapeDtypeStruct((_B, 128), jnp.float32),
        scratch_shapes=[pltpu.VMEM((_B, 128), jnp.float32)],
    )(a0, a1, hw, batch2d, w1t, b1, w2t, b2)


def kernel(x, up_index, up_orient, down_index, down_orient, batch,
           W_up_0, W_down_0, W_0, W_up_1, W_down_1, W_1, W_up_2, W_down_2, W_2,
           lin1_W, lin1_b, lin2_W, lin2_b):
    f32 = jnp.float32
    idx2 = _build_indices(up_index[0], up_orient, up_index[1],
                          down_index[0], down_orient, down_index[1])
    zeros = jnp.zeros((_N, 128), f32)

    def wcat(Wu, Wd):
        return jnp.concatenate([Wu.T, -Wu.T, Wd.T, -Wd.T], axis=1)

    layers = ((W_up_0, W_down_0, W_0), (W_up_1, W_down_1, W_1),
              (W_up_2, W_down_2, W_2))

    terms = (x,)
    hw = None
    for Wu, Wd, W in layers:
        t_tab, hw = _layer_matmul(terms, wcat(Wu, Wd), W.T)
        acc = _sc_scatter(t_tab, idx2, zeros)
        terms = (acc[0], acc[1], hw)

    w2p = jnp.zeros((128, 128), f32).at[:, :2].set(lin2_W.T)
    b2p = jnp.zeros((1, 128), f32).at[0, :2].set(lin2_b)
    out = _pool_mlp(terms[0], terms[1], terms[2], batch.reshape(_N, 1),
                    lin1_W.T, lin1_b.reshape(1, 128), w2p, b2p)
    return out[:, :2]
